# Initial kernel scaffold; baseline (speedup 1.0000x reference)
import jax
import jax.numpy as jnp
from jax.experimental import pallas as pl


def kernel(numbers, embed_table):
    raise NotImplementedError("write your pallas kernel here")


# speedup vs baseline: 1.3299x; 1.3299x over previous
import jax
import jax.numpy as jnp
from jax import lax
from jax.experimental import pallas as pl
from jax.experimental.pallas import tpu as pltpu
from jax.experimental.pallas import tpu_sc as plsc

MAX_NUM = 999999
D_MODEL = 32
QUAD = 128 // D_MODEL

NUM_CORES = 2
NUM_SUBCORES = 16
NUM_WORKERS = NUM_CORES * NUM_SUBCORES

CHUNK = 512


def _sc_gather(table_pack, idx_flat):
    b = idx_flat.shape[0]
    b_per_w = b // NUM_WORKERS
    n_chunks = b_per_w // CHUNK
    mesh = plsc.VectorSubcoreMesh(core_axis_name="c", subcore_axis_name="s")

    @pl.kernel(
        out_type=jax.ShapeDtypeStruct((b // QUAD, 128), jnp.float32),
        mesh=mesh,
        scratch_types=[
            pltpu.VMEM((CHUNK,), jnp.int32),
            pltpu.VMEM((CHUNK,), jnp.int32),
            pltpu.VMEM((CHUNK, 128), jnp.float32),
            pltpu.VMEM((CHUNK // QUAD, 128), jnp.float32),
            pltpu.SemaphoreType.DMA,
        ],
    )
    def k(table_hbm, idx_hbm, out_hbm, idx_v, q_v, quad_v, out_v, sem):
        wid = lax.axis_index("s") * NUM_CORES + lax.axis_index("c")
        base = wid * b_per_w

        @pl.loop(0, n_chunks)
        def _(g):
            off = pl.multiple_of(base + g * CHUNK, CHUNK)
            pltpu.sync_copy(idx_hbm.at[pl.ds(off, CHUNK)], idx_v)

            @pl.loop(0, CHUNK, step=16)
            def _(i):
                q_v[pl.ds(i, 16)] = jax.lax.shift_right_logical(
                    idx_v[pl.ds(i, 16)], 2
                )

            pltpu.async_copy(table_hbm.at[q_v], quad_v, sem).wait()

            @pl.loop(0, CHUNK, step=16)
            def _(r0):
                iv16 = idx_v[pl.ds(r0, 16)]
                q0 = jax.lax.shift_right_logical(r0, 2)
                for j in range(16):
                    src = (iv16[j] & 3) * D_MODEL
                    dst_r = q0 + j // 4
                    dst = (j % 4) * D_MODEL
                    out_v[dst_r, pl.ds(dst, 16)] = quad_v[r0 + j, pl.ds(src, 16)]
                    out_v[dst_r, pl.ds(dst + 16, 16)] = quad_v[
                        r0 + j, pl.ds(src + 16, 16)
                    ]

            pltpu.sync_copy(
                out_v,
                out_hbm.at[
                    pl.ds(pl.multiple_of(off // QUAD, CHUNK // QUAD), CHUNK // QUAD)
                ],
            )

    return k(table_pack, idx_flat)


def kernel(numbers, embed_table):
    batch, hist = numbers.shape
    idx_flat = numbers.reshape(batch * hist)
    table_pack = embed_table.reshape((MAX_NUM + 1) // QUAD, 128)
    out = _sc_gather(table_pack, idx_flat)
    return out.reshape(batch, hist, D_MODEL)


# speedup vs baseline: 1.3735x; 1.0328x over previous
import jax
import jax.numpy as jnp
from jax import lax
from jax.experimental import pallas as pl
from jax.experimental.pallas import tpu as pltpu
from jax.experimental.pallas import tpu_sc as plsc

MAX_NUM = 999999
D_MODEL = 32
QUAD = 128 // D_MODEL

NUM_CORES = 2
NUM_SUBCORES = 16
NUM_WORKERS = NUM_CORES * NUM_SUBCORES

NB = 8
HIST = 50
CHUNK = NB * HIST


def _sc_gather(table_pack, idx_flat, batch):
    b = idx_flat.shape[0]
    b_per_w = b // NUM_WORKERS
    nb_per_w = batch // NUM_WORKERS
    n_chunks = nb_per_w // NB
    mesh = plsc.VectorSubcoreMesh(core_axis_name="c", subcore_axis_name="s")

    @pl.kernel(
        out_type=jax.ShapeDtypeStruct((batch, HIST, D_MODEL), jnp.float32),
        mesh=mesh,
        scratch_types=[
            pltpu.VMEM((CHUNK,), jnp.int32),
            pltpu.VMEM((CHUNK,), jnp.int32),
            pltpu.VMEM((CHUNK, 128), jnp.float32),
            pltpu.VMEM((NB, HIST, D_MODEL), jnp.float32),
            pltpu.SemaphoreType.DMA,
        ],
    )
    def k(table_hbm, idx_hbm, out_hbm, idx_v, q_v, quad_v, stage_v, sem):
        wid = lax.axis_index("s") * NUM_CORES + lax.axis_index("c")
        base = wid * b_per_w
        bi_base = wid * nb_per_w

        @pl.loop(0, n_chunks)
        def _(g):
            off = pl.multiple_of(base + g * CHUNK, CHUNK)
            pltpu.sync_copy(idx_hbm.at[pl.ds(off, CHUNK)], idx_v)

            @pl.loop(0, CHUNK, step=16)
            def _(i):
                q_v[pl.ds(i, 16)] = jax.lax.shift_right_logical(
                    idx_v[pl.ds(i, 16)], 2
                )

            pltpu.async_copy(table_hbm.at[q_v], quad_v, sem).wait()

            @pl.loop(0, CHUNK, step=16)
            def _(r0):
                iv16 = idx_v[pl.ds(r0, 16)]
                for j in range(16):
                    r = r0 + j
                    src = (iv16[j] & 3) * D_MODEL
                    bb = r // HIST
                    hh = r - bb * HIST
                    stage_v[bb, hh, pl.ds(0, 16)] = quad_v[r, pl.ds(src, 16)]
                    stage_v[bb, hh, pl.ds(16, 16)] = quad_v[
                        r, pl.ds(src + 16, 16)
                    ]

            pltpu.sync_copy(stage_v, out_hbm.at[pl.ds(bi_base + g * NB, NB)])

    return k(table_pack, idx_flat)


def kernel(numbers, embed_table):
    batch, hist = numbers.shape
    idx_flat = numbers.reshape(batch * hist)
    table_pack = embed_table.reshape((MAX_NUM + 1) // QUAD, D_MODEL * QUAD)
    return _sc_gather(table_pack, idx_flat, batch)
